# baseline (speedup 1.0000x reference)
import functools

import jax
import jax.numpy as jnp
from jax.experimental import pallas as pl
from jax.experimental.pallas import tpu as pltpu

N = 10000
D = 128
H = 128
BLK = 80
NG = N // BLK


def _body(seq1_ref, seq2_ref, w0_ref, b0_ref, mskc_ref, wd_ref, bd_ref,
          sb1_ref, sb2_ref, adj1_ref, adj2_ref,
          sc1_ref, sc2_ref,
          fts1_s, fts2_s, h1_s, h2_s, csum_s, den_s):
    i = pl.program_id(0)
    nsteps = pl.num_programs(0)
    dn_t = (((1,), (1,)), ((), ()))

    @pl.when(i == 0)
    def _init():
        fts1_s[...] = jax.lax.dot_general(
            seq1_ref[...], w0_ref[...], dn_t, preferred_element_type=jnp.float32)
        fts2_s[...] = jax.lax.dot_general(
            seq2_ref[...], w0_ref[...], dn_t, preferred_element_type=jnp.float32)
        csum_s[...] = jnp.zeros_like(csum_s)
        den_s[...] = jnp.zeros_like(den_s)

    b0 = b0_ref[...]
    rows = pl.ds(i * BLK, BLK)
    h1 = jax.nn.relu(
        jnp.dot(adj1_ref[...], fts1_s[...],
                preferred_element_type=jnp.float32) + b0)
    h1_s[rows, :] = h1
    h2_s[rows, :] = jax.nn.relu(
        jnp.dot(adj2_ref[...], fts2_s[...],
                preferred_element_type=jnp.float32) + b0)

    m = mskc_ref[...]
    csum_s[...] += jnp.sum(h1 * m, axis=0, keepdims=True)
    den_s[...] += jnp.sum(m, axis=(0, 1), keepdims=True)

    @pl.when(i == nsteps - 1)
    def _finish():
        c = jax.nn.sigmoid(csum_s[...] / den_s[...])
        v = jax.lax.dot_general(c, wd_ref[...], dn_t,
                                preferred_element_type=jnp.float32)
        b = bd_ref[0, 0]
        sc1_ref[...] = jax.lax.dot_general(
            v, h1_s[...], dn_t,
            preferred_element_type=jnp.float32) + b + sb1_ref[...]
        sc2_ref[...] = jax.lax.dot_general(
            v, h2_s[...], dn_t,
            preferred_element_type=jnp.float32) + b + sb2_ref[...]


@functools.partial(jax.jit, static_argnames=("interpret",))
def _run(seq1, adj1, seq2, adj2, msk, samp_bias1, samp_bias2,
         W0, b0, disc_W, disc_b, interpret=False):
    seq1 = seq1.reshape(N, D)
    seq2 = seq2.reshape(N, D)
    adj1 = adj1.reshape(N, N)
    adj2 = adj2.reshape(N, N)
    mskc = msk.reshape(N, 1)
    sb1 = samp_bias1.reshape(1, N)
    sb2 = samp_bias2.reshape(1, N)
    b0r = b0.reshape(1, H)
    dbr = disc_b.reshape(1, 1)

    const2 = lambda i: (0, 0)
    sc1, sc2 = pl.pallas_call(
        _body,
        grid=(NG,),
        in_specs=[
            pl.BlockSpec((N, D), const2),
            pl.BlockSpec((N, D), const2),
            pl.BlockSpec((H, D), const2),
            pl.BlockSpec((1, H), const2),
            pl.BlockSpec((BLK, 1), lambda i: (i, 0)),
            pl.BlockSpec((H, H), const2),
            pl.BlockSpec((1, 1), const2),
            pl.BlockSpec((1, N), const2),
            pl.BlockSpec((1, N), const2),
            pl.BlockSpec((BLK, N), lambda i: (i, 0)),
            pl.BlockSpec((BLK, N), lambda i: (i, 0)),
        ],
        out_specs=[
            pl.BlockSpec((1, N), const2),
            pl.BlockSpec((1, N), const2),
        ],
        out_shape=[
            jax.ShapeDtypeStruct((1, N), jnp.float32),
            jax.ShapeDtypeStruct((1, N), jnp.float32),
        ],
        scratch_shapes=[
            pltpu.VMEM((N, H), jnp.float32),
            pltpu.VMEM((N, H), jnp.float32),
            pltpu.VMEM((N, H), jnp.float32),
            pltpu.VMEM((N, H), jnp.float32),
            pltpu.VMEM((1, H), jnp.float32),
            pltpu.VMEM((1, 1), jnp.float32),
        ],
        interpret=interpret,
    )(seq1, seq2, W0, b0r, mskc, disc_W, dbr, sb1, sb2, adj1, adj2)

    return jnp.concatenate([sc1, sc2], axis=1)


def kernel(seq1, adj1, seq2, adj2, sparse, msk, samp_bias1, samp_bias2,
           W0, b0, disc_W, disc_b):
    del sparse
    return _run(seq1, adj1, seq2, adj2, msk, samp_bias1, samp_bias2,
                W0, b0, jnp.asarray(disc_W), jnp.asarray(disc_b))


# speedup vs baseline: 1.0949x; 1.0949x over previous
import functools

import jax
import jax.numpy as jnp
from jax.experimental import pallas as pl
from jax.experimental.pallas import tpu as pltpu

N = 10000
D = 128
H = 128
BLK = 200
NG = N // BLK


def _body(seq1_ref, seq2_ref, w0_ref, b0_ref, mskr_ref, wd_ref, bd_ref,
          sb1_ref, sb2_ref, adj1_ref, adj2_ref,
          sc1_ref, sc2_ref,
          fts1_s, fts2_s, h1_s, h2_s):
    i = pl.program_id(0)
    nsteps = pl.num_programs(0)
    dn_t = (((1,), (1,)), ((), ()))

    @pl.when(i == 0)
    def _init():
        fts1_s[...] = jax.lax.dot_general(
            seq1_ref[...], w0_ref[...], dn_t, preferred_element_type=jnp.float32)
        fts2_s[...] = jax.lax.dot_general(
            seq2_ref[...], w0_ref[...], dn_t, preferred_element_type=jnp.float32)

    b0 = b0_ref[...]
    rows = pl.ds(i * BLK, BLK)
    h1_s[rows, :] = (jax.nn.relu(
        jnp.dot(adj1_ref[...], fts1_s[...],
                preferred_element_type=jnp.float32) + b0)).astype(jnp.bfloat16)
    h2_s[rows, :] = (jax.nn.relu(
        jnp.dot(adj2_ref[...], fts2_s[...],
                preferred_element_type=jnp.float32) + b0)).astype(jnp.bfloat16)

    @pl.when(i == nsteps - 1)
    def _finish():
        msk = mskr_ref[...].astype(jnp.bfloat16)
        csum = jnp.dot(msk, h1_s[...], preferred_element_type=jnp.float32)
        den = jnp.sum(mskr_ref[...])
        c = jax.nn.sigmoid(csum / den)
        v = jax.lax.dot_general(c, wd_ref[...], dn_t,
                                preferred_element_type=jnp.float32)
        vb = v.astype(jnp.bfloat16)
        b = bd_ref[0, 0]
        sc1_ref[...] = jax.lax.dot_general(
            vb, h1_s[...], dn_t,
            preferred_element_type=jnp.float32) + b + sb1_ref[...]
        sc2_ref[...] = jax.lax.dot_general(
            vb, h2_s[...], dn_t,
            preferred_element_type=jnp.float32) + b + sb2_ref[...]


@functools.partial(jax.jit, static_argnames=("interpret",))
def _run(seq1, adj1, seq2, adj2, msk, samp_bias1, samp_bias2,
         W0, b0, disc_W, disc_b, interpret=False):
    seq1 = seq1.reshape(N, D)
    seq2 = seq2.reshape(N, D)
    adj1 = adj1.reshape(N, N)
    adj2 = adj2.reshape(N, N)
    mskr = msk.reshape(1, N)
    sb1 = samp_bias1.reshape(1, N)
    sb2 = samp_bias2.reshape(1, N)
    b0r = b0.reshape(1, H)
    dbr = disc_b.reshape(1, 1)

    const2 = lambda i: (0, 0)
    sc1, sc2 = pl.pallas_call(
        _body,
        grid=(NG,),
        in_specs=[
            pl.BlockSpec((N, D), const2),
            pl.BlockSpec((N, D), const2),
            pl.BlockSpec((H, D), const2),
            pl.BlockSpec((1, H), const2),
            pl.BlockSpec((1, N), const2),
            pl.BlockSpec((H, H), const2),
            pl.BlockSpec((1, 1), const2),
            pl.BlockSpec((1, N), const2),
            pl.BlockSpec((1, N), const2),
            pl.BlockSpec((BLK, N), lambda i: (i, 0)),
            pl.BlockSpec((BLK, N), lambda i: (i, 0)),
        ],
        out_specs=[
            pl.BlockSpec((1, N), const2),
            pl.BlockSpec((1, N), const2),
        ],
        out_shape=[
            jax.ShapeDtypeStruct((1, N), jnp.float32),
            jax.ShapeDtypeStruct((1, N), jnp.float32),
        ],
        scratch_shapes=[
            pltpu.VMEM((N, H), jnp.float32),
            pltpu.VMEM((N, H), jnp.float32),
            pltpu.VMEM((N, H), jnp.bfloat16),
            pltpu.VMEM((N, H), jnp.bfloat16),
        ],
        interpret=interpret,
    )(seq1, seq2, W0, b0r, mskr, disc_W, dbr, sb1, sb2, adj1, adj2)

    return jnp.concatenate([sc1, sc2], axis=1)


def kernel(seq1, adj1, seq2, adj2, sparse, msk, samp_bias1, samp_bias2,
           W0, b0, disc_W, disc_b):
    del sparse
    return _run(seq1, adj1, seq2, adj2, msk, samp_bias1, samp_bias2,
                W0, b0, jnp.asarray(disc_W), jnp.asarray(disc_b))
